# Initial kernel scaffold; baseline (speedup 1.0000x reference)
import jax
import jax.numpy as jnp
from jax.experimental import pallas as pl


def kernel(x, edge_index, edge_attr, params):
    raise NotImplementedError("write your pallas kernel here")


# speedup vs baseline: 3.0326x; 3.0326x over previous
import jax
import jax.numpy as jnp
from jax import lax
from jax.experimental import pallas as pl
from jax.experimental.pallas import tpu as pltpu
from jax.experimental.pallas import tpu_sc as plsc

NC = 2
NS = 16
C = 16
K = 80
E_BLK = 4000
N_BLK = 5000


def _mesh():
    return plsc.VectorSubcoreMesh(core_axis_name="c", subcore_axis_name="s",
                                  num_cores=NC, num_subcores=NS)


def _sc_gather(x_pad, ei0, ei1):
    n_nodes = x_pad.shape[0]
    e = ei0.shape[0]
    per_w = e // (NC * NS)
    n = per_w // K

    def body(x_hbm, i0_hbm, i1_hbm, s_hbm, idx0_v, idx1_v, r0_v, r1_v,
             sem0, sem1):
        cid = lax.axis_index("c")
        sid = lax.axis_index("s")
        base = (cid * NS + sid) * per_w

        def fetch_idx(g, par):
            off = base + g * K
            pltpu.sync_copy(i0_hbm.at[pl.ds(off, K)], idx0_v.at[par])
            pltpu.sync_copy(i1_hbm.at[pl.ds(off, K)], idx1_v.at[par])

        def issue(par):
            pltpu.async_copy(x_hbm.at[idx0_v.at[par]], r0_v.at[par], sem0)
            pltpu.async_copy(x_hbm.at[idx1_v.at[par]], r1_v.at[par], sem1)

        def wait(par):
            pltpu.make_async_copy(x_hbm.at[idx0_v.at[par]], r0_v.at[par],
                                  sem0).wait()
            pltpu.make_async_copy(x_hbm.at[idx1_v.at[par]], r1_v.at[par],
                                  sem1).wait()

        fetch_idx(0, 0)
        issue(0)

        def loop_body(g, carry):
            par = lax.rem(g, 2)
            nxt = 1 - par

            @pl.when(g + 1 < n)
            def _():
                fetch_idx(g + 1, nxt)
                issue(nxt)

            wait(par)

            def ebody(i, c):
                r0_v[par, i] = r0_v[par, i] + r1_v[par, i]
                return c

            lax.fori_loop(0, K, ebody, 0)
            pltpu.sync_copy(r0_v.at[par], s_hbm.at[pl.ds(base + g * K, K), :])
            return carry

        lax.fori_loop(0, n, loop_body, 0)

    return pl.kernel(
        body,
        out_type=jax.ShapeDtypeStruct((e, C), jnp.float32),
        mesh=_mesh(),
        compiler_params=pltpu.CompilerParams(use_tc_tiling_on_sc=False),
        scratch_types=[
            pltpu.VMEM((2, K), jnp.int32),
            pltpu.VMEM((2, K), jnp.int32),
            pltpu.VMEM((2, K, C), jnp.float32),
            pltpu.VMEM((2, K, C), jnp.float32),
            pltpu.SemaphoreType.DMA,
            pltpu.SemaphoreType.DMA,
        ],
    )(x_pad, ei0, ei1)


def _sc_scatter(ea, ei0, n_nodes):
    e = ea.shape[0]
    per_w = e // (NC * NS)
    n = per_w // K
    rows_t = n_nodes // NS
    rz = 125
    nz = rows_t // rz

    def body(ea_hbm, i0_hbm, parts_hbm, idx_v, ea_v, zb_v, acc_sh,
             sem0, sem1):
        cid = lax.axis_index("c")
        sid = lax.axis_index("s")
        base = cid * (NS * per_w) + sid * per_w

        def zrow(i, c):
            zb_v[i] = jnp.zeros((C,), jnp.float32)
            return c

        lax.fori_loop(0, rz, zrow, 0)

        def zchunk(k, c):
            pltpu.sync_copy(zb_v, acc_sh.at[pl.ds(sid * rows_t + k * rz, rz), :])
            return c

        lax.fori_loop(0, nz, zchunk, 0)
        plsc.subcore_barrier()

        def fetch(g, par):
            off = base + g * K
            pltpu.sync_copy(i0_hbm.at[pl.ds(off, K)], idx_v.at[par])
            pltpu.sync_copy(ea_hbm.at[pl.ds(off, K), :], ea_v.at[par])

        fetch(0, 0)

        def loop_body(g, carry):
            par = lax.rem(g, 2)
            nxt = 1 - par

            pltpu.sync_copy(ea_v.at[par], acc_sh.at[idx_v.at[par]], add=True)

            @pl.when(g + 1 < n)
            def _():
                fetch(g + 1, nxt)

            return carry

        lax.fori_loop(0, n, loop_body, 0)
        plsc.subcore_barrier()

        def dump(k, c):
            off = sid * rows_t + k * rz
            pltpu.sync_copy(acc_sh.at[pl.ds(off, rz), :], zb_v)
            pltpu.sync_copy(zb_v, parts_hbm.at[cid, pl.ds(off, rz), :])
            return c

        lax.fori_loop(0, nz, dump, 0)

    return pl.kernel(
        body,
        out_type=jax.ShapeDtypeStruct((NC, n_nodes, C), jnp.float32),
        mesh=_mesh(),
        compiler_params=pltpu.CompilerParams(use_tc_tiling_on_sc=False),
        scratch_types=[
            pltpu.VMEM((2, K), jnp.int32),
            pltpu.VMEM((2, K, C), jnp.float32),
            pltpu.VMEM((rz, C), jnp.float32),
            pltpu.VMEM_SHARED((n_nodes, C), jnp.float32),
            pltpu.SemaphoreType.DMA,
            pltpu.SemaphoreType.DMA,
        ],
    )(ea, ei0)


def _mats(p, tag, ce_pad, pool):
    wx, bx = p[tag + "x_w"], p[tag + "x_b"]
    we, be = p[tag + "e_w"], p[tag + "e_b"]
    ox, cx = wx.shape
    oe, ce = we.shape
    ot = ox + oe
    a = jnp.zeros((ot, C), jnp.float32).at[:ox, :cx].set(wx)
    b = jnp.zeros((ot, ce_pad), jnp.float32).at[ox:, :ce].set(we)
    bias = jnp.concatenate([bx, be])
    j_n = 3 if pool else 1
    ats, bts, bs = [], [], []
    for j in range(j_n):
        aj, bj, vj = a[j::j_n], b[j::j_n], bias[j::j_n]
        o3 = aj.shape[0]
        ats.append(jnp.zeros((C, C), jnp.float32).at[:o3].set(aj).T)
        bts.append(jnp.zeros((C, ce_pad), jnp.float32).at[:o3].set(bj).T)
        bs.append(jnp.zeros((C,), jnp.float32).at[:o3].set(vj))
    return jnp.stack(ats), jnp.stack(bts), jnp.stack(bs)


def _tc_pair(u, v, at, bt, b, blk):
    m = u.shape[0]
    cv = v.shape[1]
    j_n = at.shape[0]

    def body(u_ref, v_ref, a_ref, b_ref, bias_ref, o_ref):
        uv = u_ref[...]
        vv = v_ref[...]
        acc = None
        for j in range(j_n):
            h = jnp.dot(uv, a_ref[j], preferred_element_type=jnp.float32)
            h = h + jnp.dot(vv, b_ref[j], preferred_element_type=jnp.float32)
            h = h + bias_ref[j]
            acc = h if acc is None else jnp.maximum(acc, h)
        o_ref[...] = jnp.maximum(acc, 0.0)

    return pl.pallas_call(
        body,
        grid=(m // blk,),
        in_specs=[
            pl.BlockSpec((blk, C), lambda i: (i, 0)),
            pl.BlockSpec((blk, cv), lambda i: (i, 0)),
            pl.BlockSpec(at.shape, lambda i: (0, 0, 0)),
            pl.BlockSpec(bt.shape, lambda i: (0, 0, 0)),
            pl.BlockSpec(b.shape, lambda i: (0, 0)),
        ],
        out_specs=pl.BlockSpec((blk, C), lambda i: (i, 0)),
        out_shape=jax.ShapeDtypeStruct((m, C), jnp.float32),
    )(u, v, at, bt, b)


def _tc_node(x, parts, at, bt, b, blk, want_state):
    m = x.shape[0]
    j_n = at.shape[0]

    def body(x_ref, p0_ref, p1_ref, a_ref, b_ref, bias_ref, o_ref, *rest):
        xv = x_ref[...]
        agg = p0_ref[...] + p1_ref[...]
        acc = None
        for j in range(j_n):
            h = jnp.dot(xv, a_ref[j], preferred_element_type=jnp.float32)
            h = h + jnp.dot(agg, b_ref[j], preferred_element_type=jnp.float32)
            h = h + bias_ref[j]
            acc = h if acc is None else jnp.maximum(acc, h)
        out = jnp.maximum(acc, 0.0)
        o_ref[...] = out
        if rest:
            st_ref = rest[0]

            @pl.when(pl.program_id(0) == 0)
            def _():
                st_ref[...] = jnp.zeros_like(st_ref)

            st_ref[...] += jnp.sum(out, axis=0, keepdims=True)

    out_shape = [jax.ShapeDtypeStruct((m, C), jnp.float32)]
    out_specs = [pl.BlockSpec((blk, C), lambda i: (i, 0))]
    if want_state:
        out_shape.append(jax.ShapeDtypeStruct((1, C), jnp.float32))
        out_specs.append(pl.BlockSpec((1, C), lambda i: (0, 0)))

    res = pl.pallas_call(
        body,
        grid=(m // blk,),
        in_specs=[
            pl.BlockSpec((blk, C), lambda i: (i, 0)),
            pl.BlockSpec((blk, C), lambda i: (i, 0)),
            pl.BlockSpec((blk, C), lambda i: (i, 0)),
            pl.BlockSpec(at.shape, lambda i: (0, 0, 0)),
            pl.BlockSpec(bt.shape, lambda i: (0, 0, 0)),
            pl.BlockSpec(b.shape, lambda i: (0, 0)),
        ],
        out_specs=out_specs,
        out_shape=out_shape,
    )(x, parts[0], parts[1], at, bt, b)
    return res if want_state else res[0]


def _tc_readout(x5, state, wa_t, wb_t, b1, w2_t, b2, w3_t, b3, blk):
    m = x5.shape[0]

    def body(x_ref, st_ref, wa_ref, wb_ref, b1_ref, w2_ref, b2_ref,
             w3_ref, b3_ref, o_ref):
        xv = x_ref[...]
        st = jnp.dot(st_ref[...], wa_ref[...],
                     preferred_element_type=jnp.float32)
        h1 = jnp.dot(xv, wb_ref[...], preferred_element_type=jnp.float32)
        h1 = jnp.maximum(h1 + st + b1_ref[...], 0.0)
        h2 = jnp.dot(h1, w2_ref[...], preferred_element_type=jnp.float32)
        h2 = jnp.maximum(h2 + b2_ref[...], 0.0)
        h3 = jnp.dot(h2, w3_ref[...], preferred_element_type=jnp.float32)
        o_ref[...] = jnp.maximum(h3 + b3_ref[...], 0.0)

    full = lambda arr: pl.BlockSpec(arr.shape, lambda i: (0,) * arr.ndim)
    return pl.pallas_call(
        body,
        grid=(m // blk,),
        in_specs=[
            pl.BlockSpec((blk, C), lambda i: (i, 0)),
            full(state), full(wa_t), full(wb_t), full(b1), full(w2_t),
            full(b2), full(w3_t), full(b3),
        ],
        out_specs=pl.BlockSpec((blk, 1), lambda i: (i, 0)),
        out_shape=jax.ShapeDtypeStruct((m, 1), jnp.float32),
    )(x5, state, wa_t, wb_t, b1, w2_t, b2, w3_t, b3)


def kernel(x, edge_index, edge_attr, params):
    p = params
    n_nodes = x.shape[0]
    ei0 = edge_index[0]
    ei1 = edge_index[1]

    xc = jnp.zeros((n_nodes, C), jnp.float32).at[:, : x.shape[1]].set(x)

    e_blk = E_BLK
    n_blk = N_BLK

    ea = edge_attr
    parts = None
    for l in range(1, 6):
        pool = l < 5
        ce_pad = 1 if l == 1 else C
        e_at, e_bt, e_b = _mats(p, f"e{l}", ce_pad, pool)
        s = _sc_gather(xc, ei0, ei1)
        ea = _tc_pair(s, ea, e_at, e_bt, e_b, e_blk)
        parts = _sc_scatter(ea, ei0, n_nodes)
        n_at, n_bt, n_b = _mats(p, f"n{l}", C, pool)
        if l < 5:
            xc = _tc_node(xc, parts, n_at, n_bt, n_b, n_blk, False)
        else:
            xc, state = _tc_node(xc, parts, n_at, n_bt, n_b, n_blk, True)

    fc1, fb1 = p["fc1_w"], p["fc1_b"]
    wa_t = jnp.zeros((C, 100), jnp.float32).at[:6].set(fc1[:, :6].T)
    wb_t = jnp.zeros((C, 100), jnp.float32).at[:6].set(fc1[:, 6:].T)
    q = _tc_readout(xc, state, wa_t, wb_t, fb1.reshape(1, -1),
                    p["fc2_w"].T, p["fc2_b"].reshape(1, -1),
                    p["fc3_w"].T, p["fc3_b"].reshape(1, -1), n_blk)
    return q.reshape(-1)


# speedup vs baseline: 4.4453x; 1.4659x over previous
import jax
import jax.numpy as jnp
from jax import lax
from jax.experimental import pallas as pl
from jax.experimental.pallas import tpu as pltpu
from jax.experimental.pallas import tpu_sc as plsc

NC = 2
NS = 16
C = 16
SUB = 125
NSUB = 5
K = SUB * NSUB
E_BLK = 4000
N_BLK = 5000


def _mesh():
    return plsc.VectorSubcoreMesh(core_axis_name="c", subcore_axis_name="s",
                                  num_cores=NC, num_subcores=NS)


def _sc_gather(x_pad, ei0r, ei1r):
    e = ei0r.shape[0] * SUB
    per_w = e // (NC * NS)
    n = per_w // K
    rows_per_w = per_w // SUB

    def body(x_hbm, i0_hbm, i1_hbm, s_hbm, idx0_v, idx1_v, r0_v, r1_v,
             sem0, sem1):
        cid = lax.axis_index("c")
        sid = lax.axis_index("s")
        wid = cid * NS + sid
        base = wid * per_w
        base_row = wid * rows_per_w

        def fetch_idx(g, par):
            row = base_row + g * NSUB
            pltpu.sync_copy(i0_hbm.at[pl.ds(row, NSUB), :], idx0_v.at[par])
            pltpu.sync_copy(i1_hbm.at[pl.ds(row, NSUB), :], idx1_v.at[par])

        def streams(par):
            for j in range(NSUB):
                yield (x_hbm.at[idx0_v.at[par, j]],
                       r0_v.at[par, pl.ds(j * SUB, SUB)], sem0)
                yield (x_hbm.at[idx1_v.at[par, j]],
                       r1_v.at[par, pl.ds(j * SUB, SUB)], sem1)

        def issue(par):
            for src, dst, sem in streams(par):
                pltpu.async_copy(src, dst, sem)

        def wait(par):
            for src, dst, sem in streams(par):
                pltpu.make_async_copy(src, dst, sem).wait()

        fetch_idx(0, 0)
        issue(0)

        def loop_body(g, carry):
            par = lax.rem(g, 2)
            nxt = 1 - par

            @pl.when(g + 1 < n)
            def _():
                fetch_idx(g + 1, nxt)
                issue(nxt)

            wait(par)

            def ebody(i, c):
                r0_v[par, i] = r0_v[par, i] + r1_v[par, i]
                return c

            lax.fori_loop(0, K, ebody, 0, unroll=8)
            pltpu.sync_copy(r0_v.at[par], s_hbm.at[pl.ds(base + g * K, K), :])
            return carry

        lax.fori_loop(0, n, loop_body, 0)

    return pl.kernel(
        body,
        out_type=jax.ShapeDtypeStruct((e, C), jnp.float32),
        mesh=_mesh(),
        compiler_params=pltpu.CompilerParams(use_tc_tiling_on_sc=False),
        scratch_types=[
            pltpu.VMEM((2, NSUB, SUB), jnp.int32),
            pltpu.VMEM((2, NSUB, SUB), jnp.int32),
            pltpu.VMEM((2, K, C), jnp.float32),
            pltpu.VMEM((2, K, C), jnp.float32),
            pltpu.SemaphoreType.DMA,
            pltpu.SemaphoreType.DMA,
        ],
    )(x_pad, ei0r, ei1r)


def _sc_scatter(ear, ei0r, n_nodes):
    e = ei0r.shape[0] * SUB
    per_w = e // (NC * NS)
    n = per_w // K
    rows_per_w = per_w // SUB
    rows_t = n_nodes // NS
    rz = 125
    nz = rows_t // rz

    def body(ea_hbm, i0_hbm, parts_hbm, idx_v, ea_v, zb_v, acc_sh,
             sem0, sem1):
        cid = lax.axis_index("c")
        sid = lax.axis_index("s")
        wid = cid * NS + sid
        base_row = wid * rows_per_w

        def zrow(i, c):
            zb_v[i] = jnp.zeros((C,), jnp.float32)
            return c

        lax.fori_loop(0, rz, zrow, 0)

        def zchunk(k, c):
            pltpu.sync_copy(zb_v, acc_sh.at[pl.ds(sid * rows_t + k * rz, rz), :])
            return c

        lax.fori_loop(0, nz, zchunk, 0)
        plsc.subcore_barrier()

        def fetch(g, par):
            row = base_row + g * NSUB
            pltpu.async_copy(i0_hbm.at[pl.ds(row, NSUB), :], idx_v.at[par],
                             sem0)
            pltpu.async_copy(ea_hbm.at[pl.ds(row, NSUB), :, :], ea_v.at[par],
                             sem1)

        def fwait(g, par):
            row = base_row + g * NSUB
            pltpu.make_async_copy(i0_hbm.at[pl.ds(row, NSUB), :],
                                  idx_v.at[par], sem0).wait()
            pltpu.make_async_copy(ea_hbm.at[pl.ds(row, NSUB), :, :],
                                  ea_v.at[par], sem1).wait()

        fetch(0, 0)
        fwait(0, 0)

        def loop_body(g, carry):
            par = lax.rem(g, 2)
            nxt = 1 - par

            @pl.when(g + 1 < n)
            def _():
                fetch(g + 1, nxt)

            for j in range(NSUB):
                pltpu.sync_copy(ea_v.at[par, j],
                                acc_sh.at[idx_v.at[par, j]], add=True)

            @pl.when(g + 1 < n)
            def _():
                fwait(g + 1, nxt)

            return carry

        lax.fori_loop(0, n, loop_body, 0)
        plsc.subcore_barrier()

        def dump(k, c):
            off = sid * rows_t + k * rz
            pltpu.sync_copy(acc_sh.at[pl.ds(off, rz), :], zb_v)
            pltpu.sync_copy(zb_v, parts_hbm.at[cid, pl.ds(off, rz), :])
            return c

        lax.fori_loop(0, nz, dump, 0)

    return pl.kernel(
        body,
        out_type=jax.ShapeDtypeStruct((NC, n_nodes, C), jnp.float32),
        mesh=_mesh(),
        compiler_params=pltpu.CompilerParams(use_tc_tiling_on_sc=False),
        scratch_types=[
            pltpu.VMEM((2, NSUB, SUB), jnp.int32),
            pltpu.VMEM((2, NSUB, SUB, C), jnp.float32),
            pltpu.VMEM((rz, C), jnp.float32),
            pltpu.VMEM_SHARED((n_nodes, C), jnp.float32),
            pltpu.SemaphoreType.DMA,
            pltpu.SemaphoreType.DMA,
        ],
    )(ear, ei0r)


def _mats(p, tag, ce_pad, pool):
    wx, bx = p[tag + "x_w"], p[tag + "x_b"]
    we, be = p[tag + "e_w"], p[tag + "e_b"]
    ox, cx = wx.shape
    oe, ce = we.shape
    ot = ox + oe
    a = jnp.zeros((ot, C), jnp.float32).at[:ox, :cx].set(wx)
    b = jnp.zeros((ot, ce_pad), jnp.float32).at[ox:, :ce].set(we)
    bias = jnp.concatenate([bx, be])
    j_n = 3 if pool else 1
    ats, bts, bs = [], [], []
    for j in range(j_n):
        aj, bj, vj = a[j::j_n], b[j::j_n], bias[j::j_n]
        o3 = aj.shape[0]
        ats.append(jnp.zeros((C, C), jnp.float32).at[:o3].set(aj).T)
        bts.append(jnp.zeros((C, ce_pad), jnp.float32).at[:o3].set(bj).T)
        bs.append(jnp.zeros((C,), jnp.float32).at[:o3].set(vj))
    return jnp.stack(ats), jnp.stack(bts), jnp.stack(bs)


def _tc_pair(u, v, at, bt, b, blk):
    m = u.shape[0]
    cv = v.shape[1]
    j_n = at.shape[0]

    def body(u_ref, v_ref, a_ref, b_ref, bias_ref, o_ref):
        uv = u_ref[...]
        vv = v_ref[...]
        acc = None
        for j in range(j_n):
            h = jnp.dot(uv, a_ref[j], preferred_element_type=jnp.float32)
            h = h + jnp.dot(vv, b_ref[j], preferred_element_type=jnp.float32)
            h = h + bias_ref[j]
            acc = h if acc is None else jnp.maximum(acc, h)
        o_ref[...] = jnp.maximum(acc, 0.0)

    return pl.pallas_call(
        body,
        grid=(m // blk,),
        in_specs=[
            pl.BlockSpec((blk, C), lambda i: (i, 0)),
            pl.BlockSpec((blk, cv), lambda i: (i, 0)),
            pl.BlockSpec(at.shape, lambda i: (0, 0, 0)),
            pl.BlockSpec(bt.shape, lambda i: (0, 0, 0)),
            pl.BlockSpec(b.shape, lambda i: (0, 0)),
        ],
        out_specs=pl.BlockSpec((blk, C), lambda i: (i, 0)),
        out_shape=jax.ShapeDtypeStruct((m, C), jnp.float32),
    )(u, v, at, bt, b)


def _tc_node(x, parts, at, bt, b, blk, want_state):
    m = x.shape[0]
    j_n = at.shape[0]

    def body(x_ref, p0_ref, p1_ref, a_ref, b_ref, bias_ref, o_ref, *rest):
        xv = x_ref[...]
        agg = p0_ref[...] + p1_ref[...]
        acc = None
        for j in range(j_n):
            h = jnp.dot(xv, a_ref[j], preferred_element_type=jnp.float32)
            h = h + jnp.dot(agg, b_ref[j], preferred_element_type=jnp.float32)
            h = h + bias_ref[j]
            acc = h if acc is None else jnp.maximum(acc, h)
        out = jnp.maximum(acc, 0.0)
        o_ref[...] = out
        if rest:
            st_ref = rest[0]

            @pl.when(pl.program_id(0) == 0)
            def _():
                st_ref[...] = jnp.zeros_like(st_ref)

            st_ref[...] += jnp.sum(out, axis=0, keepdims=True)

    out_shape = [jax.ShapeDtypeStruct((m, C), jnp.float32)]
    out_specs = [pl.BlockSpec((blk, C), lambda i: (i, 0))]
    if want_state:
        out_shape.append(jax.ShapeDtypeStruct((1, C), jnp.float32))
        out_specs.append(pl.BlockSpec((1, C), lambda i: (0, 0)))

    res = pl.pallas_call(
        body,
        grid=(m // blk,),
        in_specs=[
            pl.BlockSpec((blk, C), lambda i: (i, 0)),
            pl.BlockSpec((blk, C), lambda i: (i, 0)),
            pl.BlockSpec((blk, C), lambda i: (i, 0)),
            pl.BlockSpec(at.shape, lambda i: (0, 0, 0)),
            pl.BlockSpec(bt.shape, lambda i: (0, 0, 0)),
            pl.BlockSpec(b.shape, lambda i: (0, 0)),
        ],
        out_specs=out_specs,
        out_shape=out_shape,
    )(x, parts[0], parts[1], at, bt, b)
    return res if want_state else res[0]


def _tc_readout(x5, state, wa_t, wb_t, b1, w2_t, b2, w3_t, b3, blk):
    m = x5.shape[0]

    def body(x_ref, st_ref, wa_ref, wb_ref, b1_ref, w2_ref, b2_ref,
             w3_ref, b3_ref, o_ref):
        xv = x_ref[...]
        st = jnp.dot(st_ref[...], wa_ref[...],
                     preferred_element_type=jnp.float32)
        h1 = jnp.dot(xv, wb_ref[...], preferred_element_type=jnp.float32)
        h1 = jnp.maximum(h1 + st + b1_ref[...], 0.0)
        h2 = jnp.dot(h1, w2_ref[...], preferred_element_type=jnp.float32)
        h2 = jnp.maximum(h2 + b2_ref[...], 0.0)
        h3 = jnp.dot(h2, w3_ref[...], preferred_element_type=jnp.float32)
        o_ref[...] = jnp.maximum(h3 + b3_ref[...], 0.0)

    full = lambda arr: pl.BlockSpec(arr.shape, lambda i: (0,) * arr.ndim)
    return pl.pallas_call(
        body,
        grid=(m // blk,),
        in_specs=[
            pl.BlockSpec((blk, C), lambda i: (i, 0)),
            full(state), full(wa_t), full(wb_t), full(b1), full(w2_t),
            full(b2), full(w3_t), full(b3),
        ],
        out_specs=pl.BlockSpec((blk, 1), lambda i: (i, 0)),
        out_shape=jax.ShapeDtypeStruct((m, 1), jnp.float32),
    )(x5, state, wa_t, wb_t, b1, w2_t, b2, w3_t, b3)


def kernel(x, edge_index, edge_attr, params):
    p = params
    n_nodes = x.shape[0]
    n_edges = edge_index.shape[1]
    ei0r = edge_index[0].reshape(n_edges // SUB, SUB)
    ei1r = edge_index[1].reshape(n_edges // SUB, SUB)

    xc = jnp.zeros((n_nodes, C), jnp.float32).at[:, : x.shape[1]].set(x)

    e_blk = E_BLK
    n_blk = N_BLK

    ea = edge_attr
    parts = None
    for l in range(1, 6):
        pool = l < 5
        ce_pad = 1 if l == 1 else C
        e_at, e_bt, e_b = _mats(p, f"e{l}", ce_pad, pool)
        s = _sc_gather(xc, ei0r, ei1r)
        ea = _tc_pair(s, ea, e_at, e_bt, e_b, e_blk)
        parts = _sc_scatter(ea.reshape(n_edges // SUB, SUB, C), ei0r,
                            n_nodes)
        n_at, n_bt, n_b = _mats(p, f"n{l}", C, pool)
        if l < 5:
            xc = _tc_node(xc, parts, n_at, n_bt, n_b, n_blk, False)
        else:
            xc, state = _tc_node(xc, parts, n_at, n_bt, n_b, n_blk, True)

    fc1, fb1 = p["fc1_w"], p["fc1_b"]
    wa_t = jnp.zeros((C, 100), jnp.float32).at[:6].set(fc1[:, :6].T)
    wb_t = jnp.zeros((C, 100), jnp.float32).at[:6].set(fc1[:, 6:].T)
    q = _tc_readout(xc, state, wa_t, wb_t, fb1.reshape(1, -1),
                    p["fc2_w"].T, p["fc2_b"].reshape(1, -1),
                    p["fc3_w"].T, p["fc3_b"].reshape(1, -1), n_blk)
    return q.reshape(-1)


# speedup vs baseline: 15.0624x; 3.3883x over previous
import jax
import jax.numpy as jnp
from jax import lax
from jax.experimental import pallas as pl
from jax.experimental.pallas import tpu as pltpu
from jax.experimental.pallas import tpu_sc as plsc

NC = 2
NS = 16
C = 16
SUB = 125
NSUB = 5
K = SUB * NSUB
E_BLK = 2000
N_BLK = 6250


def _mesh():
    return plsc.VectorSubcoreMesh(core_axis_name="c", subcore_axis_name="s",
                                  num_cores=NC, num_subcores=NS)


def _sc_gather(x_pad, ei0r, ei1r):
    e = ei0r.shape[0] * SUB
    per_w = e // (NC * NS)
    n = per_w // K
    rows_per_w = per_w // SUB

    def body(x_hbm, i0_hbm, i1_hbm, s_hbm, idx0_v, idx1_v, r0_v, r1_v,
             sem0, sem1):
        cid = lax.axis_index("c")
        sid = lax.axis_index("s")
        wid = cid * NS + sid
        base = wid * per_w
        base_row = wid * rows_per_w

        def fetch_idx(g, par):
            row = base_row + g * NSUB
            pltpu.sync_copy(i0_hbm.at[pl.ds(row, NSUB), :], idx0_v.at[par])
            pltpu.sync_copy(i1_hbm.at[pl.ds(row, NSUB), :], idx1_v.at[par])

        def streams(par):
            for j in range(NSUB):
                yield (x_hbm.at[idx0_v.at[par, j]],
                       r0_v.at[par, pl.ds(j * SUB, SUB)], sem0)
                yield (x_hbm.at[idx1_v.at[par, j]],
                       r1_v.at[par, pl.ds(j * SUB, SUB)], sem1)

        def issue(par):
            for src, dst, sem in streams(par):
                pltpu.async_copy(src, dst, sem)

        def wait(par):
            for src, dst, sem in streams(par):
                pltpu.make_async_copy(src, dst, sem).wait()

        fetch_idx(0, 0)
        issue(0)

        def loop_body(g, carry):
            par = lax.rem(g, 2)
            nxt = 1 - par

            @pl.when(g + 1 < n)
            def _():
                fetch_idx(g + 1, nxt)
                issue(nxt)

            wait(par)

            def ebody(i, c):
                r0_v[par, i] = r0_v[par, i] + r1_v[par, i]
                return c

            lax.fori_loop(0, K, ebody, 0, unroll=8)
            pltpu.sync_copy(r0_v.at[par], s_hbm.at[pl.ds(base + g * K, K), :])
            return carry

        lax.fori_loop(0, n, loop_body, 0)

    return pl.kernel(
        body,
        out_type=jax.ShapeDtypeStruct((e, C), jnp.float32),
        mesh=_mesh(),
        compiler_params=pltpu.CompilerParams(use_tc_tiling_on_sc=False),
        scratch_types=[
            pltpu.VMEM((2, NSUB, SUB), jnp.int32),
            pltpu.VMEM((2, NSUB, SUB), jnp.int32),
            pltpu.VMEM((2, K, C), jnp.float32),
            pltpu.VMEM((2, K, C), jnp.float32),
            pltpu.SemaphoreType.DMA,
            pltpu.SemaphoreType.DMA,
        ],
    )(x_pad, ei0r, ei1r)


def _sc_scatter(ear, ei0r, n_nodes):
    e = ei0r.shape[0] * SUB
    per_w = e // (NC * NS)
    n = per_w // K
    rows_per_w = per_w // SUB
    rows_t = n_nodes // NS
    rz = 125
    nz = rows_t // rz

    def body(ea_hbm, i0_hbm, parts_hbm, idx_v, ea_v, zb_v, acc_sh,
             sem0, sem1):
        cid = lax.axis_index("c")
        sid = lax.axis_index("s")
        wid = cid * NS + sid
        base_row = wid * rows_per_w

        def zrow(i, c):
            zb_v[i] = jnp.zeros((C,), jnp.float32)
            return c

        lax.fori_loop(0, rz, zrow, 0)

        def zchunk(k, c):
            pltpu.sync_copy(zb_v, acc_sh.at[pl.ds(sid * rows_t + k * rz, rz), :])
            return c

        lax.fori_loop(0, nz, zchunk, 0)
        plsc.subcore_barrier()

        def fetch(g, par):
            row = base_row + g * NSUB
            pltpu.async_copy(i0_hbm.at[pl.ds(row, NSUB), :], idx_v.at[par],
                             sem0)
            pltpu.async_copy(ea_hbm.at[pl.ds(row, NSUB), :, :], ea_v.at[par],
                             sem1)

        def fwait(g, par):
            row = base_row + g * NSUB
            pltpu.make_async_copy(i0_hbm.at[pl.ds(row, NSUB), :],
                                  idx_v.at[par], sem0).wait()
            pltpu.make_async_copy(ea_hbm.at[pl.ds(row, NSUB), :, :],
                                  ea_v.at[par], sem1).wait()

        fetch(0, 0)
        fwait(0, 0)

        def loop_body(g, carry):
            par = lax.rem(g, 2)
            nxt = 1 - par

            @pl.when(g + 1 < n)
            def _():
                fetch(g + 1, nxt)

            for j in range(NSUB):
                pltpu.sync_copy(ea_v.at[par, j],
                                acc_sh.at[idx_v.at[par, j]], add=True)

            @pl.when(g + 1 < n)
            def _():
                fwait(g + 1, nxt)

            return carry

        lax.fori_loop(0, n, loop_body, 0)
        plsc.subcore_barrier()

        def dump(k, c):
            off = sid * rows_t + k * rz
            pltpu.sync_copy(acc_sh.at[pl.ds(off, rz), :], zb_v)
            pltpu.sync_copy(zb_v, parts_hbm.at[cid, pl.ds(off, rz), :])
            return c

        lax.fori_loop(0, nz, dump, 0)

    return pl.kernel(
        body,
        out_type=jax.ShapeDtypeStruct((NC, n_nodes, C), jnp.float32),
        mesh=_mesh(),
        compiler_params=pltpu.CompilerParams(use_tc_tiling_on_sc=False),
        scratch_types=[
            pltpu.VMEM((2, NSUB, SUB), jnp.int32),
            pltpu.VMEM((2, NSUB, SUB, C), jnp.float32),
            pltpu.VMEM((rz, C), jnp.float32),
            pltpu.VMEM_SHARED((n_nodes, C), jnp.float32),
            pltpu.SemaphoreType.DMA,
            pltpu.SemaphoreType.DMA,
        ],
    )(ear, ei0r)


def _mats(p, tag, ce_pad, pool):
    wx, bx = p[tag + "x_w"], p[tag + "x_b"]
    we, be = p[tag + "e_w"], p[tag + "e_b"]
    ox, cx = wx.shape
    oe, ce = we.shape
    ot = ox + oe
    eye8 = jnp.eye(8, dtype=jnp.float32)
    a = jnp.zeros((ot, C), jnp.float32).at[:ox, :cx].set(wx)
    b = jnp.zeros((ot, ce_pad), jnp.float32).at[ox:, :ce].set(we)
    bias = jnp.concatenate([bx, be])
    j_n = 3 if pool else 1
    ats, bts, bs = [], [], []
    for j in range(j_n):
        aj, bj, vj = a[j::j_n], b[j::j_n], bias[j::j_n]
        o3 = aj.shape[0]
        at = jnp.zeros((C, C), jnp.float32).at[:o3].set(aj).T
        bt = jnp.zeros((C, ce_pad), jnp.float32).at[:o3].set(bj).T
        ats.append(jnp.kron(eye8, at))
        bts.append(jnp.kron(eye8, bt))
        bs.append(jnp.tile(jnp.zeros((C,), jnp.float32).at[:o3].set(vj), 8))
    return jnp.stack(ats), jnp.stack(bts), jnp.stack(bs)


def _tc_pair(u, v, at, bt, b, blk):
    m = u.shape[0]
    cv = v.shape[1]
    j_n = at.shape[0]

    def body(u_ref, v_ref, a_ref, b_ref, bias_ref, o_ref):
        uv = u_ref[...]
        vv = v_ref[...]
        acc = None
        for j in range(j_n):
            h = jnp.dot(uv, a_ref[j], preferred_element_type=jnp.float32)
            h = h + jnp.dot(vv, b_ref[j], preferred_element_type=jnp.float32)
            h = h + bias_ref[j]
            acc = h if acc is None else jnp.maximum(acc, h)
        o_ref[...] = jnp.maximum(acc, 0.0)

    return pl.pallas_call(
        body,
        grid=(m // blk,),
        in_specs=[
            pl.BlockSpec((blk, 8 * C), lambda i: (i, 0)),
            pl.BlockSpec((blk, cv), lambda i: (i, 0)),
            pl.BlockSpec(at.shape, lambda i: (0, 0, 0)),
            pl.BlockSpec(bt.shape, lambda i: (0, 0, 0)),
            pl.BlockSpec(b.shape, lambda i: (0, 0)),
        ],
        out_specs=pl.BlockSpec((blk, 8 * C), lambda i: (i, 0)),
        out_shape=jax.ShapeDtypeStruct((m, 8 * C), jnp.float32),
    )(u, v, at, bt, b)


def _tc_node(x, p0, p1, at, bt, b, blk, want_state):
    m = x.shape[0]
    j_n = at.shape[0]

    def body(x_ref, p0_ref, p1_ref, a_ref, b_ref, bias_ref, o_ref, *rest):
        xv = x_ref[...]
        agg = p0_ref[...] + p1_ref[...]
        acc = None
        for j in range(j_n):
            h = jnp.dot(xv, a_ref[j], preferred_element_type=jnp.float32)
            h = h + jnp.dot(agg, b_ref[j], preferred_element_type=jnp.float32)
            h = h + bias_ref[j]
            acc = h if acc is None else jnp.maximum(acc, h)
        out = jnp.maximum(acc, 0.0)
        o_ref[...] = out
        if rest:
            st_ref = rest[0]

            @pl.when(pl.program_id(0) == 0)
            def _():
                st_ref[...] = jnp.zeros_like(st_ref)

            st_ref[...] += jnp.sum(out, axis=0, keepdims=True)

    out_shape = [jax.ShapeDtypeStruct((m, 8 * C), jnp.float32)]
    out_specs = [pl.BlockSpec((blk, 8 * C), lambda i: (i, 0))]
    if want_state:
        out_shape.append(jax.ShapeDtypeStruct((1, 8 * C), jnp.float32))
        out_specs.append(pl.BlockSpec((1, 8 * C), lambda i: (0, 0)))

    res = pl.pallas_call(
        body,
        grid=(m // blk,),
        in_specs=[
            pl.BlockSpec((blk, 8 * C), lambda i: (i, 0)),
            pl.BlockSpec((blk, 8 * C), lambda i: (i, 0)),
            pl.BlockSpec((blk, 8 * C), lambda i: (i, 0)),
            pl.BlockSpec(at.shape, lambda i: (0, 0, 0)),
            pl.BlockSpec(bt.shape, lambda i: (0, 0, 0)),
            pl.BlockSpec(b.shape, lambda i: (0, 0)),
        ],
        out_specs=out_specs,
        out_shape=out_shape,
    )(x, p0, p1, at, bt, b)
    return res if want_state else res[0]


def _tc_readout(x5, state, fold, wa_t, wb_k, b1, w2_k, b2, w3_k, b3, blk):
    m = x5.shape[0]

    def body(x_ref, st_ref, fold_ref, wa_ref, wb_ref, b1_ref, w2_ref,
             b2_ref, w3_ref, b3_ref, o_ref):
        xv = x_ref[...]
        st16 = jnp.dot(st_ref[...], fold_ref[...],
                       preferred_element_type=jnp.float32)
        c1 = jnp.dot(st16, wa_ref[...], preferred_element_type=jnp.float32)
        c1 = jnp.tile(c1 + b1_ref[...], (1, 8))
        h1 = jnp.dot(xv, wb_ref[...], preferred_element_type=jnp.float32)
        h1 = jnp.maximum(h1 + c1, 0.0)
        h2 = jnp.dot(h1, w2_ref[...], preferred_element_type=jnp.float32)
        h2 = jnp.maximum(h2 + jnp.tile(b2_ref[...], (1, 8)), 0.0)
        h3 = jnp.dot(h2, w3_ref[...], preferred_element_type=jnp.float32)
        o_ref[...] = jnp.maximum(h3 + jnp.tile(b3_ref[...], (1, 8)), 0.0)

    full = lambda arr: pl.BlockSpec(arr.shape, lambda i: (0,) * arr.ndim)
    return pl.pallas_call(
        body,
        grid=(m // blk,),
        in_specs=[
            pl.BlockSpec((blk, 8 * C), lambda i: (i, 0)),
            full(state), full(fold), full(wa_t), full(wb_k), full(b1),
            full(w2_k), full(b2), full(w3_k), full(b3),
        ],
        out_specs=pl.BlockSpec((blk, 8), lambda i: (i, 0)),
        out_shape=jax.ShapeDtypeStruct((m, 8), jnp.float32),
    )(x5, state, fold, wa_t, wb_k, b1, w2_k, b2, w3_k, b3)


def kernel(x, edge_index, edge_attr, params):
    p = params
    n_nodes = x.shape[0]
    n_edges = edge_index.shape[1]
    ei0r = edge_index[0].reshape(n_edges // SUB, SUB)
    ei1r = edge_index[1].reshape(n_edges // SUB, SUB)
    mp = n_nodes // 8
    ep = n_edges // 8

    xc = (jnp.zeros((mp, 8, C), jnp.float32)
          .at[:, :, : x.shape[1]].set(x.reshape(mp, 8, x.shape[1]))
          .reshape(mp, 8 * C))

    ea = edge_attr.reshape(ep, 8)
    state = None
    for l in range(1, 6):
        pool = l < 5
        ce_pad = 1 if l == 1 else C
        e_at, e_bt, e_b = _mats(p, f"e{l}", ce_pad, pool)
        s = _sc_gather(xc.reshape(n_nodes, C), ei0r, ei1r)
        ea = _tc_pair(s.reshape(ep, 8 * C), ea, e_at, e_bt, e_b, E_BLK)
        parts = _sc_scatter(ea.reshape(n_edges // SUB, SUB, C), ei0r,
                            n_nodes)
        pr = parts.reshape(NC, mp, 8 * C)
        n_at, n_bt, n_b = _mats(p, f"n{l}", C, pool)
        if l < 5:
            xc = _tc_node(xc, pr[0], pr[1], n_at, n_bt, n_b, N_BLK, False)
        else:
            xc, state = _tc_node(xc, pr[0], pr[1], n_at, n_bt, n_b,
                                 N_BLK, True)

    fc1, fb1 = p["fc1_w"], p["fc1_b"]
    eye8 = jnp.eye(8, dtype=jnp.float32)
    fold = jnp.tile(jnp.eye(C, dtype=jnp.float32), (8, 1))
    wa_t = jnp.zeros((C, 100), jnp.float32).at[:6].set(fc1[:, :6].T)
    wb_k = jnp.kron(eye8, jnp.zeros((C, 100), jnp.float32)
                    .at[:6].set(fc1[:, 6:].T))
    w2_k = jnp.kron(eye8, p["fc2_w"].T)
    w3_k = jnp.kron(eye8, p["fc3_w"].T)
    q = _tc_readout(xc, state, fold, wa_t, wb_k, fb1.reshape(1, -1),
                    w2_k, p["fc2_b"].reshape(1, -1),
                    w3_k, p["fc3_b"].reshape(1, -1), N_BLK)
    return q.reshape(-1)


# speedup vs baseline: 17.2747x; 1.1469x over previous
import jax
import jax.numpy as jnp
from jax import lax
from jax.experimental import pallas as pl
from jax.experimental.pallas import tpu as pltpu
from jax.experimental.pallas import tpu_sc as plsc

NC = 2
NS = 16
C = 16
SUB = 125
NSUB = 5
K = SUB * NSUB
E_BLK = 2000
N_BLK = 6250


def _mesh():
    return plsc.VectorSubcoreMesh(core_axis_name="c", subcore_axis_name="s",
                                  num_cores=NC, num_subcores=NS)


def _sc_gather(x_pad, ei0r, ei1r):
    e = ei0r.shape[0] * SUB
    per_w = e // (NC * NS)
    n = per_w // K
    rows_per_w = per_w // SUB

    def body(x_hbm, i0_hbm, i1_hbm, s_hbm, idx0_v, idx1_v, r0_v, r1_v,
             sem0, sem1, sem_i, sem_o):
        cid = lax.axis_index("c")
        sid = lax.axis_index("s")
        wid = cid * NS + sid
        base = wid * per_w
        base_row = wid * rows_per_w

        def idx_copies(g, p3):
            row = base_row + g * NSUB
            yield (i0_hbm.at[pl.ds(row, NSUB), :], idx0_v.at[p3], sem_i)
            yield (i1_hbm.at[pl.ds(row, NSUB), :], idx1_v.at[p3], sem_i)

        def fetch_idx(g, p3):
            for src, dst, sem in idx_copies(g, p3):
                pltpu.async_copy(src, dst, sem)

        def wait_idx(g, p3):
            for src, dst, sem in idx_copies(g, p3):
                pltpu.make_async_copy(src, dst, sem).wait()

        def streams(p3, par):
            for j in range(NSUB):
                yield (x_hbm.at[idx0_v.at[p3, j]],
                       r0_v.at[par, pl.ds(j * SUB, SUB)], sem0)
                yield (x_hbm.at[idx1_v.at[p3, j]],
                       r1_v.at[par, pl.ds(j * SUB, SUB)], sem1)

        def issue(p3, par):
            for src, dst, sem in streams(p3, par):
                pltpu.async_copy(src, dst, sem)

        def wait(p3, par):
            for src, dst, sem in streams(p3, par):
                pltpu.make_async_copy(src, dst, sem).wait()

        def out_copy(g, par):
            return (r0_v.at[par], s_hbm.at[pl.ds(base + g * K, K), :], sem_o)

        for src, dst, _ in idx_copies(0, 0):
            pltpu.sync_copy(src, dst)
        @pl.when(1 < n)
        def _():
            fetch_idx(1, 1)
        issue(0, 0)

        def loop_body(g, carry):
            par = lax.rem(g, 2)
            nxt = 1 - par
            p3 = lax.rem(g, 3)
            p3n = lax.rem(g + 1, 3)
            p3nn = lax.rem(g + 2, 3)

            @pl.when(g + 1 < n)
            def _():
                wait_idx(g + 1, p3n)

            @pl.when(g + 2 < n)
            def _():
                fetch_idx(g + 2, p3nn)

            @pl.when(jnp.logical_and(g >= 1, g + 1 < n))
            def _():
                src, dst, sem = out_copy(g - 1, nxt)
                pltpu.make_async_copy(src, dst, sem).wait()

            @pl.when(g + 1 < n)
            def _():
                issue(p3n, nxt)

            wait(p3, par)

            def ebody(i, c):
                r0_v[par, i] = r0_v[par, i] + r1_v[par, i]
                return c

            lax.fori_loop(0, K, ebody, 0, unroll=8)
            src, dst, sem = out_copy(g, par)
            pltpu.async_copy(src, dst, sem)
            return carry

        lax.fori_loop(0, n, loop_body, 0)
        for g_last in (n - 2, n - 1):
            src, dst, sem = out_copy(g_last, g_last % 2)
            pltpu.make_async_copy(src, dst, sem).wait()

    return pl.kernel(
        body,
        out_type=jax.ShapeDtypeStruct((e, C), jnp.float32),
        mesh=_mesh(),
        compiler_params=pltpu.CompilerParams(use_tc_tiling_on_sc=False),
        scratch_types=[
            pltpu.VMEM((3, NSUB, SUB), jnp.int32),
            pltpu.VMEM((3, NSUB, SUB), jnp.int32),
            pltpu.VMEM((2, K, C), jnp.float32),
            pltpu.VMEM((2, K, C), jnp.float32),
            pltpu.SemaphoreType.DMA,
            pltpu.SemaphoreType.DMA,
            pltpu.SemaphoreType.DMA,
            pltpu.SemaphoreType.DMA,
        ],
    )(x_pad, ei0r, ei1r)


def _sc_scatter(ear, ei0r, n_nodes):
    e = ei0r.shape[0] * SUB
    per_w = e // (NC * NS)
    n = per_w // K
    rows_per_w = per_w // SUB
    rows_t = n_nodes // NS
    rz = 125
    nz = rows_t // rz

    def body(ea_hbm, i0_hbm, parts_hbm, idx_v, ea_v, zb_v, acc_sh,
             sem0, sem1):
        cid = lax.axis_index("c")
        sid = lax.axis_index("s")
        wid = cid * NS + sid
        base_row = wid * rows_per_w

        def zrow(i, c):
            zb_v[i] = jnp.zeros((C,), jnp.float32)
            return c

        lax.fori_loop(0, rz, zrow, 0)

        def zchunk(k, c):
            pltpu.sync_copy(zb_v, acc_sh.at[pl.ds(sid * rows_t + k * rz, rz), :])
            return c

        lax.fori_loop(0, nz, zchunk, 0)
        plsc.subcore_barrier()

        def fetch(g, par):
            row = base_row + g * NSUB
            pltpu.async_copy(i0_hbm.at[pl.ds(row, NSUB), :], idx_v.at[par],
                             sem0)
            pltpu.async_copy(ea_hbm.at[pl.ds(row, NSUB), :, :], ea_v.at[par],
                             sem1)

        def fwait(g, par):
            row = base_row + g * NSUB
            pltpu.make_async_copy(i0_hbm.at[pl.ds(row, NSUB), :],
                                  idx_v.at[par], sem0).wait()
            pltpu.make_async_copy(ea_hbm.at[pl.ds(row, NSUB), :, :],
                                  ea_v.at[par], sem1).wait()

        fetch(0, 0)
        fwait(0, 0)

        def loop_body(g, carry):
            par = lax.rem(g, 2)
            nxt = 1 - par

            @pl.when(g + 1 < n)
            def _():
                fetch(g + 1, nxt)

            for j in range(NSUB):
                pltpu.sync_copy(ea_v.at[par, j],
                                acc_sh.at[idx_v.at[par, j]], add=True)

            @pl.when(g + 1 < n)
            def _():
                fwait(g + 1, nxt)

            return carry

        lax.fori_loop(0, n, loop_body, 0)
        plsc.subcore_barrier()

        def dump(k, c):
            off = sid * rows_t + k * rz
            pltpu.sync_copy(acc_sh.at[pl.ds(off, rz), :], zb_v)
            pltpu.sync_copy(zb_v, parts_hbm.at[cid, pl.ds(off, rz), :])
            return c

        lax.fori_loop(0, nz, dump, 0)

    return pl.kernel(
        body,
        out_type=jax.ShapeDtypeStruct((NC, n_nodes, C), jnp.float32),
        mesh=_mesh(),
        compiler_params=pltpu.CompilerParams(use_tc_tiling_on_sc=False),
        scratch_types=[
            pltpu.VMEM((2, NSUB, SUB), jnp.int32),
            pltpu.VMEM((2, NSUB, SUB, C), jnp.float32),
            pltpu.VMEM((rz, C), jnp.float32),
            pltpu.VMEM_SHARED((n_nodes, C), jnp.float32),
            pltpu.SemaphoreType.DMA,
            pltpu.SemaphoreType.DMA,
        ],
    )(ear, ei0r)


def _mats(p, tag, ce_pad, pool):
    wx, bx = p[tag + "x_w"], p[tag + "x_b"]
    we, be = p[tag + "e_w"], p[tag + "e_b"]
    ox, cx = wx.shape
    oe, ce = we.shape
    ot = ox + oe
    eye8 = jnp.eye(8, dtype=jnp.float32)
    a = jnp.zeros((ot, C), jnp.float32).at[:ox, :cx].set(wx)
    b = jnp.zeros((ot, ce_pad), jnp.float32).at[ox:, :ce].set(we)
    bias = jnp.concatenate([bx, be])
    j_n = 3 if pool else 1
    ats, bts, bs = [], [], []
    for j in range(j_n):
        aj, bj, vj = a[j::j_n], b[j::j_n], bias[j::j_n]
        o3 = aj.shape[0]
        at = jnp.zeros((C, C), jnp.float32).at[:o3].set(aj).T
        bt = jnp.zeros((C, ce_pad), jnp.float32).at[:o3].set(bj).T
        ats.append(jnp.kron(eye8, at))
        bts.append(jnp.kron(eye8, bt))
        bs.append(jnp.tile(jnp.zeros((C,), jnp.float32).at[:o3].set(vj), 8))
    return jnp.stack(ats), jnp.stack(bts), jnp.stack(bs)


def _tc_pair(u, v, at, bt, b, blk):
    m = u.shape[0]
    cv = v.shape[1]
    j_n = at.shape[0]

    def body(u_ref, v_ref, a_ref, b_ref, bias_ref, o_ref):
        uv = u_ref[...]
        vv = v_ref[...]
        acc = None
        for j in range(j_n):
            h = jnp.dot(uv, a_ref[j], preferred_element_type=jnp.float32)
            h = h + jnp.dot(vv, b_ref[j], preferred_element_type=jnp.float32)
            h = h + bias_ref[j]
            acc = h if acc is None else jnp.maximum(acc, h)
        o_ref[...] = jnp.maximum(acc, 0.0)

    return pl.pallas_call(
        body,
        grid=(m // blk,),
        in_specs=[
            pl.BlockSpec((blk, 8 * C), lambda i: (i, 0)),
            pl.BlockSpec((blk, cv), lambda i: (i, 0)),
            pl.BlockSpec(at.shape, lambda i: (0, 0, 0)),
            pl.BlockSpec(bt.shape, lambda i: (0, 0, 0)),
            pl.BlockSpec(b.shape, lambda i: (0, 0)),
        ],
        out_specs=pl.BlockSpec((blk, 8 * C), lambda i: (i, 0)),
        out_shape=jax.ShapeDtypeStruct((m, 8 * C), jnp.float32),
    )(u, v, at, bt, b)


def _tc_node(x, p0, p1, at, bt, b, blk, want_state):
    m = x.shape[0]
    j_n = at.shape[0]

    def body(x_ref, p0_ref, p1_ref, a_ref, b_ref, bias_ref, o_ref, *rest):
        xv = x_ref[...]
        agg = p0_ref[...] + p1_ref[...]
        acc = None
        for j in range(j_n):
            h = jnp.dot(xv, a_ref[j], preferred_element_type=jnp.float32)
            h = h + jnp.dot(agg, b_ref[j], preferred_element_type=jnp.float32)
            h = h + bias_ref[j]
            acc = h if acc is None else jnp.maximum(acc, h)
        out = jnp.maximum(acc, 0.0)
        o_ref[...] = out
        if rest:
            st_ref = rest[0]

            @pl.when(pl.program_id(0) == 0)
            def _():
                st_ref[...] = jnp.zeros_like(st_ref)

            st_ref[...] += jnp.sum(out, axis=0, keepdims=True)

    out_shape = [jax.ShapeDtypeStruct((m, 8 * C), jnp.float32)]
    out_specs = [pl.BlockSpec((blk, 8 * C), lambda i: (i, 0))]
    if want_state:
        out_shape.append(jax.ShapeDtypeStruct((1, 8 * C), jnp.float32))
        out_specs.append(pl.BlockSpec((1, 8 * C), lambda i: (0, 0)))

    res = pl.pallas_call(
        body,
        grid=(m // blk,),
        in_specs=[
            pl.BlockSpec((blk, 8 * C), lambda i: (i, 0)),
            pl.BlockSpec((blk, 8 * C), lambda i: (i, 0)),
            pl.BlockSpec((blk, 8 * C), lambda i: (i, 0)),
            pl.BlockSpec(at.shape, lambda i: (0, 0, 0)),
            pl.BlockSpec(bt.shape, lambda i: (0, 0, 0)),
            pl.BlockSpec(b.shape, lambda i: (0, 0)),
        ],
        out_specs=out_specs,
        out_shape=out_shape,
    )(x, p0, p1, at, bt, b)
    return res if want_state else res[0]


def _tc_readout(x5, state, fold, wa_t, wb_k, b1, w2_k, b2, w3_k, b3, blk):
    m = x5.shape[0]

    def body(x_ref, st_ref, fold_ref, wa_ref, wb_ref, b1_ref, w2_ref,
             b2_ref, w3_ref, b3_ref, o_ref):
        xv = x_ref[...]
        st16 = jnp.dot(st_ref[...], fold_ref[...],
                       preferred_element_type=jnp.float32)
        c1 = jnp.dot(st16, wa_ref[...], preferred_element_type=jnp.float32)
        c1 = jnp.tile(c1 + b1_ref[...], (1, 8))
        h1 = jnp.dot(xv, wb_ref[...], preferred_element_type=jnp.float32)
        h1 = jnp.maximum(h1 + c1, 0.0)
        h2 = jnp.dot(h1, w2_ref[...], preferred_element_type=jnp.float32)
        h2 = jnp.maximum(h2 + jnp.tile(b2_ref[...], (1, 8)), 0.0)
        h3 = jnp.dot(h2, w3_ref[...], preferred_element_type=jnp.float32)
        o_ref[...] = jnp.maximum(h3 + jnp.tile(b3_ref[...], (1, 8)), 0.0)

    full = lambda arr: pl.BlockSpec(arr.shape, lambda i: (0,) * arr.ndim)
    return pl.pallas_call(
        body,
        grid=(m // blk,),
        in_specs=[
            pl.BlockSpec((blk, 8 * C), lambda i: (i, 0)),
            full(state), full(fold), full(wa_t), full(wb_k), full(b1),
            full(w2_k), full(b2), full(w3_k), full(b3),
        ],
        out_specs=pl.BlockSpec((blk, 8), lambda i: (i, 0)),
        out_shape=jax.ShapeDtypeStruct((m, 8), jnp.float32),
    )(x5, state, fold, wa_t, wb_k, b1, w2_k, b2, w3_k, b3)


def kernel(x, edge_index, edge_attr, params):
    p = params
    n_nodes = x.shape[0]
    n_edges = edge_index.shape[1]
    ei0r = edge_index[0].reshape(n_edges // SUB, SUB)
    ei1r = edge_index[1].reshape(n_edges // SUB, SUB)
    mp = n_nodes // 8
    ep = n_edges // 8

    xc = (jnp.zeros((mp, 8, C), jnp.float32)
          .at[:, :, : x.shape[1]].set(x.reshape(mp, 8, x.shape[1]))
          .reshape(mp, 8 * C))

    ea = edge_attr.reshape(ep, 8)
    state = None
    for l in range(1, 6):
        pool = l < 5
        ce_pad = 1 if l == 1 else C
        e_at, e_bt, e_b = _mats(p, f"e{l}", ce_pad, pool)
        s = _sc_gather(xc.reshape(n_nodes, C), ei0r, ei1r)
        ea = _tc_pair(s.reshape(ep, 8 * C), ea, e_at, e_bt, e_b, E_BLK)
        parts = _sc_scatter(ea.reshape(n_edges // SUB, SUB, C), ei0r,
                            n_nodes)
        pr = parts.reshape(NC, mp, 8 * C)
        n_at, n_bt, n_b = _mats(p, f"n{l}", C, pool)
        if l < 5:
            xc = _tc_node(xc, pr[0], pr[1], n_at, n_bt, n_b, N_BLK, False)
        else:
            xc, state = _tc_node(xc, pr[0], pr[1], n_at, n_bt, n_b,
                                 N_BLK, True)

    fc1, fb1 = p["fc1_w"], p["fc1_b"]
    eye8 = jnp.eye(8, dtype=jnp.float32)
    fold = jnp.tile(jnp.eye(C, dtype=jnp.float32), (8, 1))
    wa_t = jnp.zeros((C, 100), jnp.float32).at[:6].set(fc1[:, :6].T)
    wb_k = jnp.kron(eye8, jnp.zeros((C, 100), jnp.float32)
                    .at[:6].set(fc1[:, 6:].T))
    w2_k = jnp.kron(eye8, p["fc2_w"].T)
    w3_k = jnp.kron(eye8, p["fc3_w"].T)
    q = _tc_readout(xc, state, fold, wa_t, wb_k, fb1.reshape(1, -1),
                    w2_k, p["fc2_b"].reshape(1, -1),
                    w3_k, p["fc3_b"].reshape(1, -1), N_BLK)
    return q.reshape(-1)
